# baseline (speedup 1.0000x reference)
import functools

import jax
import jax.numpy as jnp
from jax import lax
from jax.experimental import pallas as pl
from jax.experimental.pallas import tpu as pltpu
from jax.experimental.pallas import tpu_sc as plsc

BATCH = 16384
EMBED = 32
HIDDEN = 64

_INFO = plsc.get_sparse_core_info()
_NC = _INFO.num_cores
_NS = _INFO.num_subcores
_NW = _NC * _NS
_B_PER_W = BATCH // _NW
_CHUNK = 128
_NCHUNK = _B_PER_W // _CHUNK


def _sc_gather(uidx2d, midx2d, user_table, movie_table):
  mesh = plsc.VectorSubcoreMesh(core_axis_name="c", subcore_axis_name="s")

  @functools.partial(
      pl.kernel,
      mesh=mesh,
      out_type=[
          jax.ShapeDtypeStruct((BATCH, EMBED), jnp.float32),
          jax.ShapeDtypeStruct((BATCH, EMBED), jnp.float32),
      ],
      scratch_types=[
          pltpu.VMEM((_NCHUNK, _CHUNK), jnp.int32),
          pltpu.VMEM((_NCHUNK, _CHUNK), jnp.int32),
          pltpu.VMEM((_B_PER_W, EMBED), jnp.float32),
          pltpu.VMEM((_B_PER_W, EMBED), jnp.float32),
          pltpu.SemaphoreType.DMA,
      ],
      compiler_params=pltpu.CompilerParams(use_tc_tiling_on_sc=False),
  )
  def k(uidx_hbm, midx_hbm, utab_hbm, mtab_hbm, out_u, out_m,
        uidx_v, midx_v, urows_v, mrows_v, sem):
    wid = lax.axis_index("s") * _NC + lax.axis_index("c")
    row0 = wid * _NCHUNK
    pltpu.sync_copy(uidx_hbm.at[pl.ds(row0, _NCHUNK)], uidx_v)
    pltpu.sync_copy(midx_hbm.at[pl.ds(row0, _NCHUNK)], midx_v)
    copies = []
    for j in range(_NCHUNK):
      copies.append(pltpu.async_copy(
          utab_hbm.at[uidx_v.at[j]],
          urows_v.at[pl.ds(j * _CHUNK, _CHUNK)], sem))
      copies.append(pltpu.async_copy(
          mtab_hbm.at[midx_v.at[j]],
          mrows_v.at[pl.ds(j * _CHUNK, _CHUNK)], sem))
    for c in copies:
      c.wait()
    base = wid * _B_PER_W
    pltpu.sync_copy(urows_v, out_u.at[pl.ds(base, _B_PER_W)])
    pltpu.sync_copy(mrows_v, out_m.at[pl.ds(base, _B_PER_W)])

  return k(uidx2d, midx2d, user_table, movie_table)


_BLK = 2048


def _mlp_body(u_ref, m_ref, w1_ref, b1_ref, w2_ref, b2_ref, o_ref):
  w1 = w1_ref[...]
  h = (jnp.dot(u_ref[...], w1[:EMBED, :], preferred_element_type=jnp.float32)
       + jnp.dot(m_ref[...], w1[EMBED:, :], preferred_element_type=jnp.float32)
       + b1_ref[...])
  h = jnp.maximum(h, 0.0)
  o_ref[...] = jnp.sum(h * w2_ref[...], axis=1) + b2_ref[...]


def _tc_mlp(u_g, m_g, W1, b1, W2, b2):
  grid = (BATCH // _BLK,)
  return pl.pallas_call(
      _mlp_body,
      grid=grid,
      in_specs=[
          pl.BlockSpec((_BLK, EMBED), lambda i: (i, 0)),
          pl.BlockSpec((_BLK, EMBED), lambda i: (i, 0)),
          pl.BlockSpec((2 * EMBED, HIDDEN), lambda i: (0, 0)),
          pl.BlockSpec((1, HIDDEN), lambda i: (0, 0)),
          pl.BlockSpec((1, HIDDEN), lambda i: (0, 0)),
          pl.BlockSpec((1,), lambda i: (0,)),
      ],
      out_specs=pl.BlockSpec((_BLK,), lambda i: (i,)),
      out_shape=jax.ShapeDtypeStruct((BATCH,), jnp.float32),
  )(u_g, m_g, W1, b1.reshape(1, HIDDEN), W2.reshape(1, HIDDEN), b2)


def kernel(user_idx, movie_idx, user_table, movie_table, W1, b1, W2, b2):
  uidx2d = user_idx.astype(jnp.int32).reshape(BATCH // _CHUNK, _CHUNK)
  midx2d = movie_idx.astype(jnp.int32).reshape(BATCH // _CHUNK, _CHUNK)
  u_g, m_g = _sc_gather(uidx2d, midx2d, user_table, movie_table)
  return _tc_mlp(u_g, m_g, W1, b1, W2, b2)


# speedup vs baseline: 1.4058x; 1.4058x over previous
import functools

import jax
import jax.numpy as jnp
from jax import lax
from jax.experimental import pallas as pl
from jax.experimental.pallas import tpu as pltpu
from jax.experimental.pallas import tpu_sc as plsc

BATCH = 16384
EMBED = 32
HIDDEN = 64
NUM_U = 1000000
NUM_M = 100000

_NC = 2
_NS = 16
_NW = _NC * _NS
_LANES = 128

_W = 512
_NGRP = BATCH // 16
_DUMMY = BATCH
_OUT_ROWS = BATCH + 256


def _ranges(nrows):
  tiles = -(-nrows // _LANES)
  tpw = -(-tiles // _NW)
  range_cols = tpw * _LANES
  nch = -(-range_cols // _W)
  clamp = (tiles - _W // _LANES) * _LANES
  return range_cols, nch, clamp


_RANGE_U, _NCH_U, _CLAMP_U = _ranges(NUM_U)
_RANGE_M, _NCH_M, _CLAMP_M = _ranges(NUM_M)


def _sc_scan_gather(ut_t, mt_t, uidx, midx):
  mesh = plsc.VectorSubcoreMesh(core_axis_name="c", subcore_axis_name="s")

  @functools.partial(
      pl.kernel,
      mesh=mesh,
      out_type=[
          jax.ShapeDtypeStruct((_OUT_ROWS, _LANES), jnp.float32),
          jax.ShapeDtypeStruct((_OUT_ROWS, _LANES), jnp.float32),
      ],
      scratch_types=[
          pltpu.VMEM((BATCH,), jnp.int32),
          pltpu.VMEM((BATCH,), jnp.int32),
          pltpu.VMEM((BATCH,), jnp.int32),
          pltpu.VMEM((BATCH,), jnp.int32),
          pltpu.VMEM((BATCH,), jnp.int32),
          pltpu.VMEM((EMBED, _W), jnp.float32),
          pltpu.VMEM((_LANES, _LANES), jnp.float32),
          pltpu.VMEM((1, _LANES), jnp.int32),
          pltpu.SemaphoreType.DMA,
      ],
      compiler_params=pltpu.CompilerParams(
          use_tc_tiling_on_sc=True, needs_layout_passes=False),
  )
  def k(ut_hbm, mt_hbm, uidx_hbm, midx_hbm, out_u, out_m,
        idx_v, huh, hup, hmh, hmp, buf, stg_f, stg_p, sem):
    wid = lax.axis_index("s") * _NC + lax.axis_index("c")
    iota = lax.iota(jnp.int32, 16)
    zeros16 = jnp.zeros((16,), jnp.float32)
    dummy16 = jnp.full((16,), _DUMMY, jnp.int32)

    for r in range(_LANES):
      for t in range(EMBED // 16, _LANES // 16):
        stg_f[r, pl.ds(t * 16, 16)] = zeros16
    for t in range(_LANES // 16):
      stg_p[0, pl.ds(t * 16, 16)] = dummy16

    def phase_a(idx_hbm, hh, hp, lo, range_cols):
      pltpu.sync_copy(idx_hbm, idx_v)
      hi = lo + range_cols

      def body(g, cnt):
        off = pl.multiple_of(g * 16, 8)
        v = idx_v[pl.ds(off, 16)]
        m = (v >= lo) & (v < hi)
        mi = jnp.where(m, 1, 0)
        dest = cnt + plsc.cumsum(mi) - 1
        plsc.store_scatter(hh, [dest], v, mask=m)
        plsc.store_scatter(hp, [dest], g * 16 + iota, mask=m)
        return cnt + lax.reduce_sum(mi, axes=(0,))

      return lax.fori_loop(0, _NGRP, body, jnp.int32(0))

    def phase_b(tbl_hbm, hh, hp, cnt, lo, nch, clamp, out):
      ngrp = (cnt + 15) // 16

      def flush(valid):
        pltpu.async_copy(stg_f, out.at[stg_p.at[0]], sem).wait()
        for t in range(_LANES // 16):
          stg_p[0, pl.ds(t * 16, 16)] = dummy16

      def chunk(kk, s_idx):
        c0 = jnp.minimum(lo + kk * _W, clamp)
        c0 = pl.multiple_of(c0, _LANES)
        pltpu.sync_copy(tbl_hbm.at[:, pl.ds(c0, _W)], buf)

        def group(j, s_idx):
          goff = pl.multiple_of(j * 16, 8)
          h = hh[pl.ds(goff, 16)]
          p = hp[pl.ds(goff, 16)]
          valid = (goff + iota) < cnt
          m0 = jnp.where((h >= c0) & (h < c0 + _W) & valid, 1, 0)

          def w_cond(carry):
            m, _ = carry
            return lax.reduce_sum(m, axes=(0,)) > 0

          def w_body(carry):
            m, s_idx = carry
            mb = m > 0
            lane = plsc.all_reduce_ffs(mb)
            onehot = iota == lane
            hv = lax.reduce_sum(jnp.where(onehot, h, 0), axes=(0,))
            pv = lax.reduce_sum(jnp.where(onehot, p, 0), axes=(0,))
            colv = jnp.full((16,), hv - c0, jnp.int32)
            g1 = plsc.load_gather(buf, [iota, colv])
            g2 = plsc.load_gather(buf, [iota + 16, colv])
            stg_f[s_idx, pl.ds(0, 16)] = g1
            stg_f[s_idx, pl.ds(16, 16)] = g2
            plsc.store_scatter(
                stg_p, [jnp.zeros((16,), jnp.int32), jnp.full((16,), s_idx, jnp.int32)],
                jnp.full((16,), pv, jnp.int32), mask=iota == 0)
            s_idx = s_idx + 1

            @pl.when(s_idx == _LANES)
            def _():
              flush(s_idx)

            s_idx = jnp.where(s_idx == _LANES, 0, s_idx)
            return jnp.where(onehot, 0, m), s_idx

          _, s_idx = lax.while_loop(w_cond, w_body, (m0, s_idx))
          return s_idx

        return lax.fori_loop(0, ngrp, group, s_idx)

      s_idx = lax.fori_loop(0, nch, chunk, jnp.int32(0))

      @pl.when(s_idx > 0)
      def _():
        flush(s_idx)

    cnt_u = phase_a(uidx_hbm, huh, hup, wid * _RANGE_U, _RANGE_U)
    cnt_m = phase_a(midx_hbm, hmh, hmp, wid * _RANGE_M, _RANGE_M)
    phase_b(ut_hbm, huh, hup, cnt_u, wid * _RANGE_U, _NCH_U, _CLAMP_U, out_u)
    phase_b(mt_hbm, hmh, hmp, cnt_m, wid * _RANGE_M, _NCH_M, _CLAMP_M, out_m)

  return k(ut_t, mt_t, uidx, midx)


_BLK = 2048


def _mlp_body(u_ref, m_ref, w1u_ref, w1m_ref, b1_ref, w2_ref, b2_ref, o_ref):
  h = (jnp.dot(u_ref[...], w1u_ref[...], preferred_element_type=jnp.float32)
       + jnp.dot(m_ref[...], w1m_ref[...], preferred_element_type=jnp.float32)
       + b1_ref[...])
  h = jnp.maximum(h, 0.0)
  o_ref[...] = jnp.sum(h * w2_ref[...], axis=1) + b2_ref[...]


def _tc_mlp(u_g, m_g, W1u, W1m, b1, W2, b2):
  grid = (BATCH // _BLK,)
  return pl.pallas_call(
      _mlp_body,
      grid=grid,
      in_specs=[
          pl.BlockSpec((_BLK, _LANES), lambda i: (i, 0)),
          pl.BlockSpec((_BLK, _LANES), lambda i: (i, 0)),
          pl.BlockSpec((_LANES, HIDDEN), lambda i: (0, 0)),
          pl.BlockSpec((_LANES, HIDDEN), lambda i: (0, 0)),
          pl.BlockSpec((1, HIDDEN), lambda i: (0, 0)),
          pl.BlockSpec((1, HIDDEN), lambda i: (0, 0)),
          pl.BlockSpec((1,), lambda i: (0,)),
      ],
      out_specs=pl.BlockSpec((_BLK,), lambda i: (i,)),
      out_shape=jax.ShapeDtypeStruct((BATCH,), jnp.float32),
  )(u_g, m_g, W1u, W1m, b1.reshape(1, HIDDEN), W2.reshape(1, HIDDEN), b2)


def kernel(user_idx, movie_idx, user_table, movie_table, W1, b1, W2, b2):
  uidx = user_idx.astype(jnp.int32)
  midx = movie_idx.astype(jnp.int32)
  u_g, m_g = _sc_scan_gather(user_table.T, movie_table.T, uidx, midx)
  W1u = jnp.zeros((_LANES, HIDDEN), jnp.float32).at[:EMBED].set(W1[:EMBED])
  W1m = jnp.zeros((_LANES, HIDDEN), jnp.float32).at[:EMBED].set(W1[EMBED:])
  return _tc_mlp(u_g, m_g, W1u, W1m, b1, W2, b2)


# speedup vs baseline: 2.2160x; 1.5764x over previous
import functools

import jax
import jax.numpy as jnp
from jax import lax
from jax.experimental import pallas as pl
from jax.experimental.pallas import tpu as pltpu
from jax.experimental.pallas import tpu_sc as plsc

BATCH = 16384
EMBED = 32
HIDDEN = 64
NUM_U = 1000000
NUM_M = 100000

_NC = 2
_NS = 16
_NW = _NC * _NS
_LANES = 128

_W = 1024
_NGRP = BATCH // 16
_DUMMY = BATCH
_OUT_ROWS = BATCH + 256
_STG = 64


def _ranges(nrows):
  tiles = -(-nrows // _LANES)
  tpw = -(-tiles // _NW)
  range_cols = tpw * _LANES
  nch = -(-range_cols // _W)
  clamp = (tiles - _W // _LANES) * _LANES
  return range_cols, nch, clamp


_RANGE_U, _NCH_U, _CLAMP_U = _ranges(NUM_U)
_RANGE_M, _NCH_M, _CLAMP_M = _ranges(NUM_M)


def _sc_scan_gather(ut_t, mt_t, uidx, midx):
  mesh = plsc.VectorSubcoreMesh(core_axis_name="c", subcore_axis_name="s")

  @functools.partial(
      pl.kernel,
      mesh=mesh,
      out_type=[
          jax.ShapeDtypeStruct((_OUT_ROWS, _LANES), jnp.float32),
          jax.ShapeDtypeStruct((_OUT_ROWS, _LANES), jnp.float32),
      ],
      scratch_types=[
          pltpu.VMEM((BATCH,), jnp.int32),
          pltpu.VMEM((BATCH,), jnp.int32),
          pltpu.VMEM((BATCH,), jnp.int32),
          pltpu.VMEM((2, EMBED, _W), jnp.float32),
          pltpu.VMEM((_STG, _LANES), jnp.float32),
          pltpu.VMEM((1, _STG), jnp.int32),
          pltpu.SemaphoreType.DMA,
          pltpu.SemaphoreType.DMA,
          pltpu.SemaphoreType.DMA,
      ],
      compiler_params=pltpu.CompilerParams(
          use_tc_tiling_on_sc=True, needs_layout_passes=False),
  )
  def k(ut_hbm, mt_hbm, uidx_hbm, midx_hbm, out_u, out_m,
        idx_v, hup, hmp, bufs, stg_f, stg_p, sem0, sem1, semf):
    wid = lax.axis_index("s") * _NC + lax.axis_index("c")
    iota = lax.iota(jnp.int32, 16)
    zeros16 = jnp.zeros((16,), jnp.float32)
    dummy16 = jnp.full((16,), _DUMMY, jnp.int32)

    for r in range(_STG):
      for t in range(EMBED // 16, _LANES // 16):
        stg_f[r, pl.ds(t * 16, 16)] = zeros16
    for t in range(_STG // 16):
      stg_p[0, pl.ds(t * 16, 16)] = dummy16

    def phase_a(idx_hbm, hp, lo, range_cols):
      pltpu.sync_copy(idx_hbm, idx_v)
      hi = lo + range_cols

      def body(g, cnt):
        off = pl.multiple_of(g * 16, 8)
        v = idx_v[pl.ds(off, 16)]
        m = (v >= lo) & (v < hi)
        mi = jnp.where(m, 1, 0)
        tot = lax.reduce_sum(mi, axes=(0,))

        @pl.when(tot > 0)
        def _():
          dest = cnt + plsc.cumsum(mi) - 1
          plsc.store_scatter(hp, [dest], g * 16 + iota, mask=m)

        return cnt + tot

      return lax.fori_loop(0, _NGRP, body, jnp.int32(0))

    def reset_stg_p():
      for t in range(_STG // 16):
        stg_p[0, pl.ds(t * 16, 16)] = dummy16

    def phase_b(tbl_hbm, hp, cnt, lo, nch, clamp, out, sems):
      ngrp = (cnt + 15) // 16

      def c0_of(kk):
        c0 = jnp.minimum(lo + kk * _W, clamp)
        return pl.multiple_of(c0, _LANES)

      def group(buf_ref, c0):
        def body(j, s_idx):
          goff = pl.multiple_of(j * 16, 8)
          p = hp[pl.ds(goff, 16)] & (BATCH - 1)
          h = plsc.load_gather(idx_v, [p])
          valid = (goff + iota) < cnt
          m0 = jnp.where((h >= c0) & (h < c0 + _W) & valid, 1, 0)

          def w_cond(carry):
            m, _ = carry
            return lax.reduce_sum(m, axes=(0,)) > 0

          def w_body(carry):
            m, s_idx = carry
            mb = m > 0
            lane = plsc.all_reduce_ffs(mb)
            onehot = iota == lane
            hv = lax.reduce_sum(jnp.where(onehot, h, 0), axes=(0,))
            pv = lax.reduce_sum(jnp.where(onehot, p, 0), axes=(0,))
            colv = jnp.full((16,), hv - c0, jnp.int32)
            g1 = plsc.load_gather(buf_ref, [iota, colv])
            g2 = plsc.load_gather(buf_ref, [iota + 16, colv])
            stg_f[s_idx, pl.ds(0, 16)] = g1
            stg_f[s_idx, pl.ds(16, 16)] = g2
            plsc.store_scatter(
                stg_p,
                [jnp.zeros((16,), jnp.int32),
                 jnp.full((16,), s_idx, jnp.int32)],
                jnp.full((16,), pv, jnp.int32), mask=iota == 0)
            s_idx = s_idx + 1

            @pl.when(s_idx == _STG)
            def _():
              pltpu.async_copy(stg_f, out.at[stg_p.at[0]], semf).wait()
              reset_stg_p()

            s_idx = jnp.where(s_idx == _STG, 0, s_idx)
            return jnp.where(onehot, 0, m), s_idx

          _, s_idx = lax.while_loop(w_cond, w_body, (m0, s_idx))
          return s_idx

        return body

      s_idx = jnp.int32(0)
      pend = pltpu.async_copy(
          tbl_hbm.at[:, pl.ds(c0_of(0), _W)], bufs.at[0], sems[0])
      for kk in range(nch):
        b = kk % 2
        nxt = None
        if kk + 1 < nch:
          nxt = pltpu.async_copy(
              tbl_hbm.at[:, pl.ds(c0_of(kk + 1), _W)],
              bufs.at[(kk + 1) % 2], sems[(kk + 1) % 2])
        pend.wait()
        s_idx = lax.fori_loop(0, ngrp, group(bufs.at[b], c0_of(kk)), s_idx)
        pend = nxt

      @pl.when(s_idx > 0)
      def _():
        pltpu.async_copy(stg_f, out.at[stg_p.at[0]], semf).wait()
        reset_stg_p()

    cnt_u = phase_a(uidx_hbm, hup, wid * _RANGE_U, _RANGE_U)
    phase_b(ut_hbm, hup, cnt_u, wid * _RANGE_U, _NCH_U, _CLAMP_U, out_u,
            (sem0, sem1))
    cnt_m = phase_a(midx_hbm, hmp, wid * _RANGE_M, _RANGE_M)
    phase_b(mt_hbm, hmp, cnt_m, wid * _RANGE_M, _NCH_M, _CLAMP_M, out_m,
            (sem0, sem1))

  return k(ut_t, mt_t, uidx, midx)


_BLK = 2048


def _mlp_body(u_ref, m_ref, w1u_ref, w1m_ref, b1_ref, w2_ref, b2_ref, o_ref):
  h = (jnp.dot(u_ref[...], w1u_ref[...], preferred_element_type=jnp.float32)
       + jnp.dot(m_ref[...], w1m_ref[...], preferred_element_type=jnp.float32)
       + b1_ref[...])
  h = jnp.maximum(h, 0.0)
  o_ref[...] = jnp.sum(h * w2_ref[...], axis=1) + b2_ref[...]


def _tc_mlp(u_g, m_g, W1u, W1m, b1, W2, b2):
  grid = (BATCH // _BLK,)
  return pl.pallas_call(
      _mlp_body,
      grid=grid,
      in_specs=[
          pl.BlockSpec((_BLK, _LANES), lambda i: (i, 0)),
          pl.BlockSpec((_BLK, _LANES), lambda i: (i, 0)),
          pl.BlockSpec((_LANES, HIDDEN), lambda i: (0, 0)),
          pl.BlockSpec((_LANES, HIDDEN), lambda i: (0, 0)),
          pl.BlockSpec((1, HIDDEN), lambda i: (0, 0)),
          pl.BlockSpec((1, HIDDEN), lambda i: (0, 0)),
          pl.BlockSpec((1,), lambda i: (0,)),
      ],
      out_specs=pl.BlockSpec((_BLK,), lambda i: (i,)),
      out_shape=jax.ShapeDtypeStruct((BATCH,), jnp.float32),
  )(u_g, m_g, W1u, W1m, b1.reshape(1, HIDDEN), W2.reshape(1, HIDDEN), b2)


def kernel(user_idx, movie_idx, user_table, movie_table, W1, b1, W2, b2):
  uidx = user_idx.astype(jnp.int32)
  midx = movie_idx.astype(jnp.int32)
  u_g, m_g = _sc_scan_gather(user_table.T, movie_table.T, uidx, midx)
  W1u = jnp.zeros((_LANES, HIDDEN), jnp.float32).at[:EMBED].set(W1[:EMBED])
  W1m = jnp.zeros((_LANES, HIDDEN), jnp.float32).at[:EMBED].set(W1[EMBED:])
  return _tc_mlp(u_g, m_g, W1u, W1m, b1, W2, b2)
